# baseline (speedup 1.0000x reference)
import functools

import jax
import jax.numpy as jnp
from jax import lax
from jax.experimental import pallas as pl
from jax.experimental.pallas import tpu as pltpu
from jax.experimental.pallas import tpu_sc as plsc

B, L, D, V = 4096, 200, 64, 1024
N = B * L
PW = 128
NC, NS = 2, 16
NW = NC * NS
BPW = B // NW
PER_W = BPW * L
C1, C2 = 128, L - 128
NBUF = 2
NGRP = BPW // NBUF

_mesh = plsc.VectorSubcoreMesh(core_axis_name="c", subcore_axis_name="s")


@functools.partial(
    pl.kernel,
    mesh=_mesh,
    compiler_params=pltpu.CompilerParams(use_tc_tiling_on_sc=True),
    out_type=jax.ShapeDtypeStruct((B, L, D), jnp.float32),
    scratch_types=[
        pltpu.VMEM((PER_W,), jnp.int32),
        pltpu.VMEM((PER_W,), jnp.int32),
        pltpu.VMEM((NBUF, L, D), jnp.float32),
        pltpu.VMEM_SHARED((V, D), jnp.float32),
        pltpu.VMEM_SHARED((V, D), jnp.float32),
        pltpu.SemaphoreType.DMA((NBUF,)),
        pltpu.SemaphoreType.DMA((NBUF,)),
        pltpu.SemaphoreType.DMA((NBUF,)),
    ],
)
def _emb_sum_kernel(i1_hbm, i2_hbm, t1_hbm, t2_hbm, out_hbm,
                    idx1, idx2, rbuf, sh1, sh2, sem1, sem2, sem3):
    wid = lax.axis_index("s") * NC + lax.axis_index("c")
    base = wid * PER_W

    @pl.when(lax.axis_index("s") == 0)
    def _stage_tables():
        pltpu.sync_copy(t1_hbm, sh1)
        pltpu.sync_copy(t2_hbm, sh2)

    pltpu.sync_copy(i1_hbm.at[pl.ds(base, PER_W)], idx1)
    pltpu.sync_copy(i2_hbm.at[pl.ds(base, PER_W)], idx2)
    plsc.subcore_barrier()

    def gather_parts(table, idx, g, b):
        goff = (g * NBUF + b) * L
        return [
            (table.at[idx.at[pl.ds(goff, C1)]], rbuf.at[b, pl.ds(0, C1)]),
            (table.at[idx.at[pl.ds(goff + C1, C2)]], rbuf.at[b, pl.ds(C1, C2)]),
        ]

    def fire_gathers(table, idx, g, b, sem, add):
        for src, dst in gather_parts(table, idx, g, b):
            pltpu.async_copy(src, dst, sem.at[b], add=add)

    def wait_gathers(table, idx, g, b, sem):
        for src, dst in gather_parts(table, idx, g, b):
            pltpu.make_async_copy(src, dst, sem.at[b]).wait()

    def store_pair(g, b):
        return (rbuf.at[b],
                out_hbm.at[wid * BPW + g * NBUF + b, :, pl.ds(0, D)])

    def group(g, carry):
        for b in range(NBUF):
            @pl.when(g > 0)
            def _drain(b=b):
                src, dst = store_pair(g, b)
                pltpu.make_async_copy(src, dst, sem3.at[b]).wait()
            fire_gathers(sh1, idx1, g, b, sem1, False)
        for b in range(NBUF):
            wait_gathers(sh1, idx1, g, b, sem1)
            fire_gathers(sh2, idx2, g, b, sem2, True)
        for b in range(NBUF):
            wait_gathers(sh2, idx2, g, b, sem2)
            src, dst = store_pair(g, b)
            pltpu.async_copy(src, dst, sem3.at[b])
        return carry

    lax.fori_loop(0, NGRP, group, 0)

    for b in range(NBUF):
        src, dst = store_pair(NGRP - 1, b)
        pltpu.make_async_copy(src, dst, sem3.at[b]).wait()


def kernel(pos_1, pos_2, table1, table2):
    zrow = jnp.zeros((1, D), jnp.float32)
    t1a = jnp.concatenate([zrow, table1[: V - 1]], axis=0)
    t2a = jnp.concatenate([zrow, table2[: V - 1]], axis=0)
    i1 = pos_1.reshape(N).astype(jnp.int32)
    i2 = pos_2.reshape(N).astype(jnp.int32)
    return _emb_sum_kernel(i1, i2, t1a, t2a)


# speedup vs baseline: 1.4634x; 1.4634x over previous
import functools

import jax
import jax.numpy as jnp
from jax import lax
from jax.experimental import pallas as pl
from jax.experimental.pallas import tpu as pltpu
from jax.experimental.pallas import tpu_sc as plsc

B, L, D, V = 4096, 200, 64, 1024
N = B * L
PW = 128
NC, NS = 2, 16
NW = NC * NS
BPW = B // NW
PER_W = BPW * L
C1, C2 = 128, L - 128
NBUF = 8
NGRP = BPW // NBUF
GIDX = NBUF * L

_mesh = plsc.VectorSubcoreMesh(core_axis_name="c", subcore_axis_name="s")


@functools.partial(
    pl.kernel,
    mesh=_mesh,
    compiler_params=pltpu.CompilerParams(use_tc_tiling_on_sc=False),
    out_type=jax.ShapeDtypeStruct((B, L, PW), jnp.float32),
    scratch_types=[
        pltpu.VMEM((2, GIDX), jnp.int32),
        pltpu.VMEM((2, GIDX), jnp.int32),
        pltpu.VMEM((NBUF, L, D), jnp.float32),
        pltpu.VMEM_SHARED((V, D), jnp.float32),
        pltpu.VMEM_SHARED((V, D), jnp.float32),
        pltpu.SemaphoreType.DMA((2,)),
        pltpu.SemaphoreType.DMA((NBUF,)),
        pltpu.SemaphoreType.DMA((NBUF,)),
        pltpu.SemaphoreType.DMA((NBUF,)),
    ],
)
def _emb_sum_kernel(i1_hbm, i2_hbm, t1_hbm, t2_hbm, out_hbm,
                    idx1, idx2, rbuf, sh1, sh2, semi, sem1, sem2, sem3):
    wid = lax.axis_index("s") * NC + lax.axis_index("c")
    base = wid * PER_W

    @pl.when(lax.axis_index("s") == 0)
    def _stage_tables():
        pltpu.sync_copy(t1_hbm, sh1)
        pltpu.sync_copy(t2_hbm, sh2)

    def idx_pairs(g, slot):
        goff = base + g * GIDX
        return [
            (i1_hbm.at[pl.ds(goff, GIDX)], idx1.at[slot]),
            (i2_hbm.at[pl.ds(goff, GIDX)], idx2.at[slot]),
        ]

    def fire_idx(g, slot):
        for src, dst in idx_pairs(g, slot):
            pltpu.async_copy(src, dst, semi.at[slot])

    def wait_idx(g, slot):
        for src, dst in idx_pairs(g, slot):
            pltpu.make_async_copy(src, dst, semi.at[slot]).wait()

    fire_idx(0, 0)
    plsc.subcore_barrier()

    def gather_parts(table, idx, slot, b):
        goff = b * L
        return [
            (table.at[idx.at[slot, pl.ds(goff, C1)]],
             rbuf.at[b, pl.ds(0, C1)]),
            (table.at[idx.at[slot, pl.ds(goff + C1, C2)]],
             rbuf.at[b, pl.ds(C1, C2)]),
        ]

    def fire_gathers(table, idx, slot, b, sem, add):
        for src, dst in gather_parts(table, idx, slot, b):
            pltpu.async_copy(src, dst, sem.at[b], add=add)

    def wait_gathers(table, idx, slot, b, sem):
        for src, dst in gather_parts(table, idx, slot, b):
            pltpu.make_async_copy(src, dst, sem.at[b]).wait()

    def store_pair(g, b):
        return (rbuf.at[b],
                out_hbm.at[wid * BPW + g * NBUF + b, :, pl.ds(0, D)])

    def group(g, carry):
        slot = g % 2
        wait_idx(g, slot)
        @pl.when(g + 1 < NGRP)
        def _prefetch():
            fire_idx(g + 1, (g + 1) % 2)
        for b in range(NBUF):
            @pl.when(g > 0)
            def _drain(b=b):
                src, dst = store_pair(g, b)
                pltpu.make_async_copy(src, dst, sem3.at[b]).wait()
            fire_gathers(sh1, idx1, slot, b, sem1, False)
        for b in range(NBUF):
            wait_gathers(sh1, idx1, slot, b, sem1)
            fire_gathers(sh2, idx2, slot, b, sem2, True)
        for b in range(NBUF):
            wait_gathers(sh2, idx2, slot, b, sem2)
            src, dst = store_pair(g, b)
            pltpu.async_copy(src, dst, sem3.at[b])
        return carry

    lax.fori_loop(0, NGRP, group, 0)

    for b in range(NBUF):
        src, dst = store_pair(NGRP - 1, b)
        pltpu.make_async_copy(src, dst, sem3.at[b]).wait()


def kernel(pos_1, pos_2, table1, table2):
    zrow = jnp.zeros((1, D), jnp.float32)
    t1a = jnp.concatenate([zrow, table1[: V - 1]], axis=0)
    t2a = jnp.concatenate([zrow, table2[: V - 1]], axis=0)
    i1 = pos_1.reshape(N).astype(jnp.int32)
    i2 = pos_2.reshape(N).astype(jnp.int32)
    out = _emb_sum_kernel(i1, i2, t1a, t2a)
    return out[..., :D]


# speedup vs baseline: 1.4759x; 1.0086x over previous
import functools

import jax
import jax.numpy as jnp
from jax import lax
from jax.experimental import pallas as pl
from jax.experimental.pallas import tpu as pltpu
from jax.experimental.pallas import tpu_sc as plsc

B, L, D, V = 4096, 200, 64, 1024
N = B * L
PW = 128
NC, NS = 2, 16
NW = NC * NS
BPW = B // NW
PER_W = BPW * L
C1, C2 = 128, L - 128
NBUF = 8
NGRP = BPW // NBUF
GIDX = NBUF * L

_mesh = plsc.VectorSubcoreMesh(core_axis_name="c", subcore_axis_name="s")


@functools.partial(
    pl.kernel,
    mesh=_mesh,
    compiler_params=pltpu.CompilerParams(use_tc_tiling_on_sc=False),
    out_type=jax.ShapeDtypeStruct((B, L, PW), jnp.float32),
    scratch_types=[
        pltpu.VMEM((2, NBUF, L), jnp.int32),
        pltpu.VMEM((2, NBUF, L), jnp.int32),
        pltpu.VMEM((NBUF, L, D), jnp.float32),
        pltpu.VMEM_SHARED((V, D), jnp.float32),
        pltpu.VMEM_SHARED((V, D), jnp.float32),
        pltpu.SemaphoreType.DMA((2,)),
        pltpu.SemaphoreType.DMA((NBUF,)),
        pltpu.SemaphoreType.DMA((NBUF,)),
        pltpu.SemaphoreType.DMA((NBUF,)),
    ],
)
def _emb_sum_kernel(i1_hbm, i2_hbm, t1_hbm, t2_hbm, out_hbm,
                    idx1, idx2, rbuf, sh1, sh2, semi, sem1, sem2, sem3):
    wid = lax.axis_index("s") * NC + lax.axis_index("c")
    base = wid * PER_W

    @pl.when(lax.axis_index("s") == 0)
    def _stage_tables():
        pltpu.sync_copy(t1_hbm, sh1)
        pltpu.sync_copy(t2_hbm, sh2)

    def idx_pairs(g, slot):
        row0 = wid * BPW + g * NBUF
        return [
            (i1_hbm.at[pl.ds(row0, NBUF)], idx1.at[slot]),
            (i2_hbm.at[pl.ds(row0, NBUF)], idx2.at[slot]),
        ]

    def fire_idx(g, slot):
        for src, dst in idx_pairs(g, slot):
            pltpu.async_copy(src, dst, semi.at[slot])

    def wait_idx(g, slot):
        for src, dst in idx_pairs(g, slot):
            pltpu.make_async_copy(src, dst, semi.at[slot]).wait()

    fire_idx(0, 0)
    plsc.subcore_barrier()

    def gather_parts(table, idx, slot, b):
        return [
            (table.at[idx.at[slot, b, pl.ds(0, C1)]],
             rbuf.at[b, pl.ds(0, C1)]),
            (table.at[idx.at[slot, b, pl.ds(C1, C2)]],
             rbuf.at[b, pl.ds(C1, C2)]),
        ]

    def fire_gathers(table, idx, slot, b, sem, add):
        for src, dst in gather_parts(table, idx, slot, b):
            pltpu.async_copy(src, dst, sem.at[b], add=add)

    def wait_gathers(table, idx, slot, b, sem):
        for src, dst in gather_parts(table, idx, slot, b):
            pltpu.make_async_copy(src, dst, sem.at[b]).wait()

    def store_pair(g, b):
        return (rbuf.at[b],
                out_hbm.at[wid * BPW + g * NBUF + b, :, pl.ds(0, D)])

    def group(g, carry):
        slot = g % 2
        wait_idx(g, slot)
        @pl.when(g + 1 < NGRP)
        def _prefetch():
            fire_idx(g + 1, (g + 1) % 2)
        for b in range(NBUF):
            @pl.when(g > 0)
            def _drain(b=b):
                src, dst = store_pair(g, b)
                pltpu.make_async_copy(src, dst, sem3.at[b]).wait()
            fire_gathers(sh1, idx1, slot, b, sem1, False)
        for b in range(NBUF):
            wait_gathers(sh1, idx1, slot, b, sem1)
            fire_gathers(sh2, idx2, slot, b, sem2, True)
        for b in range(NBUF):
            wait_gathers(sh2, idx2, slot, b, sem2)
            src, dst = store_pair(g, b)
            pltpu.async_copy(src, dst, sem3.at[b])
        return carry

    lax.fori_loop(0, NGRP, group, 0)

    for b in range(NBUF):
        src, dst = store_pair(NGRP - 1, b)
        pltpu.make_async_copy(src, dst, sem3.at[b]).wait()


def kernel(pos_1, pos_2, table1, table2):
    zrow = jnp.zeros((1, D), jnp.float32)
    t1a = jnp.concatenate([zrow, table1[: V - 1]], axis=0)
    t2a = jnp.concatenate([zrow, table2[: V - 1]], axis=0)
    i1 = pos_1.astype(jnp.int32)
    i2 = pos_2.astype(jnp.int32)
    out = _emb_sum_kernel(i1, i2, t1a, t2a)
    return out[..., :D]
